# Initial kernel scaffold; baseline (speedup 1.0000x reference)
import jax
import jax.numpy as jnp
from jax.experimental import pallas as pl


def kernel(input_ids, embedding_table):
    raise NotImplementedError("write your pallas kernel here")


# speedup vs baseline: 1.4690x; 1.4690x over previous
import functools

import jax
import jax.numpy as jnp
from jax import lax
from jax.experimental import pallas as pl
from jax.experimental.pallas import tpu as pltpu
from jax.experimental.pallas import tpu_sc as plsc

EMBED_DIM = 32

_NC = 2
_NS = 16
_NW = _NC * _NS

_TOT = 16384 * 20
_PER_W = _TOT // _NW
_CHUNK = 512
_NCHUNK = _PER_W // _CHUNK
_NBUF = 4

_mesh = plsc.VectorSubcoreMesh(core_axis_name="c", subcore_axis_name="s")


@functools.partial(
    pl.kernel,
    mesh=_mesh,
    out_type=jax.ShapeDtypeStruct((_TOT, EMBED_DIM), jnp.float32),
    scratch_types=(
        [pltpu.VMEM((_PER_W,), jnp.int32)]
        + [pltpu.VMEM((_CHUNK, EMBED_DIM), jnp.float32) for _ in range(_NBUF)]
        + [pltpu.SemaphoreType.DMA for _ in range(2 * _NBUF)]
    ),
    compiler_params=pltpu.CompilerParams(use_tc_tiling_on_sc=False),
)
def _gather_kernel(ids_hbm, table_hbm, out_hbm, idx_v, *bufs_sems):
    rows = bufs_sems[:_NBUF]
    gsem = bufs_sems[_NBUF:2 * _NBUF]
    wsem = bufs_sems[2 * _NBUF:]

    wid = lax.axis_index("s") * _NC + lax.axis_index("c")
    base = wid * _PER_W

    pltpu.sync_copy(ids_hbm.at[pl.ds(base, _PER_W)], idx_v)

    def start_gather(i, b):
        return pltpu.async_copy(
            table_hbm.at[idx_v.at[pl.ds(i * _CHUNK, _CHUNK)]], rows[b], gsem[b])

    def start_write(i, b):
        return pltpu.async_copy(
            rows[b], out_hbm.at[pl.ds(base + i * _CHUNK, _CHUNK)], wsem[b])

    g = [None] * _NBUF
    w = [None] * _NBUF
    for i in range(min(_NBUF, _NCHUNK)):
        g[i] = start_gather(i, i)
    for i in range(_NCHUNK):
        b = i % _NBUF
        g[b].wait()
        w[b] = start_write(i, b)
        j = i + _NBUF
        if j < _NCHUNK:
            w[b].wait()
            g[b] = start_gather(j, b)
        else:
            w[b].wait()


def kernel(input_ids, embedding_table):
    flat = input_ids.reshape(-1).astype(jnp.int32)
    out = _gather_kernel(flat, embedding_table)
    out = out.reshape(input_ids.shape + (EMBED_DIM,))
    return (out, embedding_table)
